# Initial kernel scaffold; baseline (speedup 1.0000x reference)
import jax
import jax.numpy as jnp
from jax.experimental import pallas as pl


def kernel(x, edge_index, edge_attr, c1_init, c1_w, c1_root, c1_bias, bn_g, bn_b, c2_init, c2_w, c2_root, c2_bias):
    raise NotImplementedError("write your pallas kernel here")


# speedup vs baseline: 67.8317x; 67.8317x over previous
import functools

import jax
import jax.numpy as jnp
from jax import lax
from jax.experimental import pallas as pl
from jax.experimental.pallas import tpu as pltpu
from jax.experimental.pallas import tpu_sc as plsc

N = 100000
E = 1600000
HID = 16
K = 3
NUM_LAYERS = 4

NPAD = 102400
GB = 4096
GRID = NPAD // GB
R128 = NPAD // 128
SB = GB // 128
NC = 2
NS = 16
TR = NPAD // NS
E2 = E // NC
TE = E2 // NS
W = 80
NWIN = TE // W
EPAD = 1638400
ER = EPAD // 128
ER_SC = ER // NC
ER_TILE = ER_SC // NS
CH = 8
NCHUNK = ER_TILE // CH
F32 = jnp.float32
HIGHEST = lax.Precision.HIGHEST


def _mesh():
    return plsc.VectorSubcoreMesh(core_axis_name="c", subcore_axis_name="s",
                                  num_cores=NC, num_subcores=NS)


ZB = 128


def _zero_zb2(zb):
    def z(i, _):
        zb[i, :] = jnp.zeros((HID,), F32)
        return 0
    lax.fori_loop(0, ZB, z, 0)


def _zero_zb1(zb):
    def z(i, _):
        zb[pl.ds(i * 16, 16)] = jnp.zeros((16,), F32)
        return 0
    lax.fori_loop(0, ZB // 16, z, 0)


@functools.lru_cache(maxsize=None)
def _deg_fn():
    def body(dst_h, ew_h, out_h, dst_v, ew_v, zb, acc):
        c = lax.axis_index("c")
        s = lax.axis_index("s")
        _zero_zb1(zb)
        for r in range(TR // ZB):
            pltpu.sync_copy(zb, acc.at[pl.ds(s * TR + r * ZB, ZB)])
        plsc.subcore_barrier()
        base0 = c * E2 + s * TE

        def w_body(w, _):
            b = base0 + w * W
            pltpu.sync_copy(dst_h.at[pl.ds(b, W)], dst_v)
            pltpu.sync_copy(ew_h.at[pl.ds(b, W)], ew_v)
            pltpu.sync_copy(ew_v, acc.at[dst_v], add=True)
            return 0

        lax.fori_loop(0, NWIN, w_body, 0)
        plsc.subcore_barrier()
        sl = pl.ds(s * TR, TR)
        pltpu.sync_copy(acc.at[sl], out_h.at[c, sl])

    return pl.kernel(
        body,
        out_type=jax.ShapeDtypeStruct((NC, NPAD), F32),
        mesh=_mesh(),
        compiler_params=pltpu.CompilerParams(use_tc_tiling_on_sc=False),
        scratch_types=[
            pltpu.VMEM((W,), jnp.int32),
            pltpu.VMEM((W,), F32),
            pltpu.VMEM((ZB,), F32),
            pltpu.VMEM_SHARED((NPAD,), F32),
        ],
    )


@functools.lru_cache(maxsize=None)
def _spmm_fn():
    def body(src_h, dst_h, ew_h, hk_h, out_h, srcb, dstb, ewb, upd, zb,
             acc, isem, gsem0, gsem1, ssem0, ssem1):
        c = lax.axis_index("c")
        s = lax.axis_index("s")
        _zero_zb2(zb)
        for r in range(TR // ZB):
            pltpu.sync_copy(zb, acc.at[pl.ds(s * TR + r * ZB, ZB)])
        plsc.subcore_barrier()
        base = c * ER_SC + s * ER_TILE
        gsem = (gsem0, gsem1)
        ssem = (ssem0, ssem1)

        def idx_issue(q, p):
            rows = pl.ds(base + q * CH, CH)
            pltpu.async_copy(src_h.at[rows], srcb.at[p], isem)
            pltpu.async_copy(dst_h.at[rows], dstb.at[p], isem)
            pltpu.async_copy(ew_h.at[rows], ewb.at[p], isem)

        def idx_wait(p):
            pltpu.make_async_copy(src_h.at[pl.ds(0, CH)], srcb.at[p],
                                  isem).wait()
            pltpu.make_async_copy(src_h.at[pl.ds(0, CH)], dstb.at[p],
                                  isem).wait()
            pltpu.make_async_copy(ew_h.at[pl.ds(0, CH)], ewb.at[p],
                                  isem).wait()

        def sdrain(u):
            pltpu.make_async_copy(hk_h.at[pl.ds(0, 128)], upd.at[u],
                                  ssem[u]).wait()

        idx_issue(0, 0)

        def chunk(q, _):
            p = q % 2
            idx_wait(p)

            @pl.when(q < NCHUNK - 1)
            def _issue_next_idx():
                idx_issue(q + 1, 1 - p)

            @pl.when(q > 0)
            def _drain_prev_s0():
                sdrain(0)

            pltpu.async_copy(hk_h.at[srcb.at[p, 0]], upd.at[0], gsem0)
            for sub in range(CH):
                u = sub % 2
                if sub == 0:
                    @pl.when(q > 0)
                    def _drain_prev_s1():
                        sdrain(1)
                elif sub < CH - 1:
                    sdrain(1 - u)
                if sub < CH - 1:
                    pltpu.async_copy(hk_h.at[srcb.at[p, sub + 1]],
                                     upd.at[1 - u], gsem[1 - u])
                pltpu.make_async_copy(hk_h.at[pl.ds(0, 128)], upd.at[u],
                                      gsem[u]).wait()

                def m_body(j, _):
                    ewv = ewb[p, sub, pl.ds(j * 16, 16)]
                    for t in range(16):
                        jj = j * 16 + t
                        upd[u, jj, :] = upd[u, jj, :] * ewv[t]
                    return 0

                lax.fori_loop(0, 8, m_body, 0)
                pltpu.async_copy(upd.at[u], acc.at[dstb.at[p, sub]],
                                 ssem[u], add=True)
            return 0

        lax.fori_loop(0, NCHUNK, chunk, 0)
        sdrain(0)
        sdrain(1)
        plsc.subcore_barrier()
        sl = pl.ds(s * TR, TR)
        pltpu.sync_copy(acc.at[sl], out_h.at[c, sl])

    return pl.kernel(
        body,
        out_type=jax.ShapeDtypeStruct((NC, NPAD, HID), F32),
        mesh=_mesh(),
        compiler_params=pltpu.CompilerParams(use_tc_tiling_on_sc=False),
        scratch_types=[
            pltpu.VMEM((2, CH, 128), jnp.int32),
            pltpu.VMEM((2, CH, 128), jnp.int32),
            pltpu.VMEM((2, CH, 128), F32),
            pltpu.VMEM((2, 128, HID), F32),
            pltpu.VMEM((ZB, HID), F32),
            pltpu.VMEM_SHARED((NPAD, HID), F32),
            pltpu.SemaphoreType.DMA,
            pltpu.SemaphoreType.DMA,
            pltpu.SemaphoreType.DMA,
            pltpu.SemaphoreType.DMA,
            pltpu.SemaphoreType.DMA,
        ],
    )


@functools.lru_cache(maxsize=None)
def _spmv_fn(C):
    def body(src_h, dst_h, ew_h, *rest):
        xs = rest[:C]
        out_h = rest[C]
        src_v, dst_v, ew_v, xg, upd, zb = rest[C + 1:C + 7]
        vsh = rest[C + 7:C + 7 + C]
        acc = rest[C + 7 + C:C + 7 + 2 * C]
        gsem = rest[C + 7 + 2 * C]
        c = lax.axis_index("c")
        s = lax.axis_index("s")
        sl = pl.ds(s * TR, TR)
        for ch in range(C):
            pltpu.sync_copy(xs[ch].at[sl], vsh[ch].at[sl])
        _zero_zb1(zb)
        for ch in range(C):
            for r in range(TR // ZB):
                pltpu.sync_copy(zb, acc[ch].at[pl.ds(s * TR + r * ZB, ZB)])
        plsc.subcore_barrier()
        base0 = c * E2 + s * TE

        def w_body(w, _):
            b = base0 + w * W
            pltpu.sync_copy(src_h.at[pl.ds(b, W)], src_v)
            pltpu.sync_copy(dst_h.at[pl.ds(b, W)], dst_v)
            pltpu.sync_copy(ew_h.at[pl.ds(b, W)], ew_v)
            for ch in range(C):
                pltpu.async_copy(vsh[ch].at[src_v], xg, gsem).wait()
                for t in range(W // 16):
                    s16 = pl.ds(t * 16, 16)
                    upd[s16] = xg[s16] * ew_v[s16]
                pltpu.sync_copy(upd, acc[ch].at[dst_v], add=True)
            return 0

        lax.fori_loop(0, NWIN, w_body, 0)
        plsc.subcore_barrier()
        for ch in range(C):
            pltpu.sync_copy(acc[ch].at[sl], out_h.at[c, ch, sl])

    return pl.kernel(
        body,
        out_type=jax.ShapeDtypeStruct((NC, C, NPAD), F32),
        mesh=_mesh(),
        compiler_params=pltpu.CompilerParams(use_tc_tiling_on_sc=False),
        scratch_types=(
            [pltpu.VMEM((W,), jnp.int32),
             pltpu.VMEM((W,), jnp.int32),
             pltpu.VMEM((W,), F32),
             pltpu.VMEM((W,), F32),
             pltpu.VMEM((W,), F32),
             pltpu.VMEM((ZB,), F32)]
            + [pltpu.VMEM_SHARED((NPAD,), F32) for _ in range(2 * C)]
            + [pltpu.SemaphoreType.DMA]
        ),
    )


def _sds(shape):
    return jax.ShapeDtypeStruct(shape, F32)


def _full_spec(shape):
    rank = len(shape)
    return pl.BlockSpec(shape, lambda i, _r=rank: (0,) * _r)


def _t1(degp2, x2):
    def body(degp_ref, x_ref, dinv2_ref, xt2_ref):
        deg = degp_ref[0] + degp_ref[1]
        pos = deg > 0
        safe = jnp.where(pos, deg, 1.0)
        dinv = jnp.where(pos, lax.rsqrt(safe), 0.0)
        dinv2_ref[...] = dinv
        xt2_ref[...] = dinv * x_ref[...]

    return pl.pallas_call(
        body,
        grid=(GRID,),
        in_specs=[
            pl.BlockSpec((NC, SB, 128), lambda i: (0, i, 0)),
            pl.BlockSpec((SB, 128), lambda i: (i, 0)),
        ],
        out_specs=[
            pl.BlockSpec((SB, 128), lambda i: (i, 0)),
            pl.BlockSpec((SB, 128), lambda i: (i, 0)),
        ],
        out_shape=[_sds((R128, 128)), _sds((R128, 128))],
    )(degp2, x2)


def _t2a(sp2, dinv2):
    def body(sp_ref, dinv2_ref, a_ref):
        a_ref[...] = dinv2_ref[...] * (sp_ref[0, 0] + sp_ref[1, 0])

    return pl.pallas_call(
        body,
        grid=(GRID,),
        in_specs=[
            pl.BlockSpec((NC, 1, SB, 128), lambda i: (0, 0, i, 0)),
            pl.BlockSpec((SB, 128), lambda i: (i, 0)),
        ],
        out_specs=[pl.BlockSpec((SB, 128), lambda i: (i, 0))],
        out_shape=[_sds((R128, 128))],
    )(sp2, dinv2)[0]


def _t2(arep, drep, xrep, c1_init, c1_w, c1_root, c1_bias):
    def body(arep_ref, drep_ref, xrep_ref, ci_ref, cw_ref, cr_ref,
             cb_ref, out_ref):
        ar = arep_ref[...]
        xr = xrep_ref[...]
        dr = drep_ref[...]
        for k in range(K):
            o = ar * ci_ref[k] + xr * cr_ref[0, k] + cb_ref[0, k]
            o = jnp.maximum(o, 0.0)
            hk = jnp.dot(o, cw_ref[0, k], precision=HIGHEST)
            out_ref[k] = dr * hk

    return pl.pallas_call(
        body,
        grid=(GRID,),
        in_specs=[
            pl.BlockSpec((GB, HID), lambda i: (i, 0)),
            pl.BlockSpec((GB, HID), lambda i: (i, 0)),
            pl.BlockSpec((GB, HID), lambda i: (i, 0)),
            _full_spec((K, 1, HID)),
            _full_spec((1, K, HID, HID)),
            _full_spec((1, K, 1, HID)),
            _full_spec((1, K, 1, HID)),
        ],
        out_specs=[pl.BlockSpec((K, GB, HID), lambda i: (0, i, 0))],
        out_shape=[_sds((K, NPAD, HID))],
    )(arep, drep, xrep, c1_init, c1_w, c1_root, c1_bias)[0]


def _t3(p0, p1, p2, drep, xrep, c1_w, c1_root, c1_bias):
    def body(p0_ref, p1_ref, p2_ref, drep_ref, xrep_ref, cw_ref, cr_ref,
             cb_ref, out_ref):
        ps = (p0_ref, p1_ref, p2_ref)
        xr = xrep_ref[...]
        dr = drep_ref[...]
        for k in range(K):
            agg = dr * (ps[k][0] + ps[k][1])
            o = agg + xr * cr_ref[0, k] + cb_ref[0, k]
            o = jnp.maximum(o, 0.0)
            out_ref[k] = dr * jnp.dot(o, cw_ref[0, k], precision=HIGHEST)

    part_spec = pl.BlockSpec((NC, GB, HID), lambda i: (0, i, 0))
    return pl.pallas_call(
        body,
        grid=(GRID,),
        in_specs=[
            part_spec, part_spec, part_spec,
            pl.BlockSpec((GB, HID), lambda i: (i, 0)),
            pl.BlockSpec((GB, HID), lambda i: (i, 0)),
            _full_spec((1, K, HID, HID)),
            _full_spec((1, K, 1, HID)),
            _full_spec((1, K, 1, HID)),
        ],
        out_specs=[pl.BlockSpec((K, GB, HID), lambda i: (0, i, 0))],
        out_shape=[_sds((K, NPAD, HID))],
    )(p0, p1, p2, drep, xrep, c1_w, c1_root, c1_bias)[0]


def _t3c(p0, p1, p2, drep, xrep, c1_root, c1_bias):
    def body(p0_ref, p1_ref, p2_ref, drep_ref, xrep_ref, cr_ref, cb_ref,
             h_ref, sum_ref, sq_ref):
        i = pl.program_id(0)
        ps = (p0_ref, p1_ref, p2_ref)
        xr = xrep_ref[...]
        dr = drep_ref[...]
        acc = jnp.zeros((GB, HID), F32)
        for k in range(K):
            o = dr * (ps[k][0] + ps[k][1]) + xr * cr_ref[0, k] + cb_ref[0, k]
            acc = acc + jnp.maximum(o, 0.0)
        h = acc * (1.0 / K)
        h_ref[...] = h
        node = i * GB + lax.broadcasted_iota(jnp.int32, (GB, 1), 0)
        hm = jnp.where(node < N, h, 0.0)
        sum_ref[...] = jnp.broadcast_to(
            jnp.sum(hm, axis=0, keepdims=True), (8, HID))[None]
        sq_ref[...] = jnp.broadcast_to(
            jnp.sum(hm * hm, axis=0, keepdims=True), (8, HID))[None]

    part_spec = pl.BlockSpec((NC, GB, HID), lambda i: (0, i, 0))
    return pl.pallas_call(
        body,
        grid=(GRID,),
        in_specs=[
            part_spec, part_spec, part_spec,
            pl.BlockSpec((GB, HID), lambda i: (i, 0)),
            pl.BlockSpec((GB, HID), lambda i: (i, 0)),
            _full_spec((1, K, 1, HID)),
            _full_spec((1, K, 1, HID)),
        ],
        out_specs=[
            pl.BlockSpec((GB, HID), lambda i: (i, 0)),
            pl.BlockSpec((1, 8, HID), lambda i: (i, 0, 0)),
            pl.BlockSpec((1, 8, HID), lambda i: (i, 0, 0)),
        ],
        out_shape=[_sds((NPAD, HID)), _sds((GRID, 8, HID)),
                   _sds((GRID, 8, HID))],
    )(p0, p1, p2, drep, xrep, c1_root, c1_bias)


def _t4(h, sums, sqs, gam, bet, drep, c2_init, c2_root, c2_bias):
    def body(h_ref, sums_ref, sqs_ref, gam_ref, bet_ref, drep_ref, ci_ref,
             cr_ref, cb_ref, vt_ref, r2_ref):
        mu = jnp.sum(jnp.sum(sums_ref[...], axis=0), axis=0,
                     keepdims=True) * (1.0 / (8 * N))
        ex2 = jnp.sum(jnp.sum(sqs_ref[...], axis=0), axis=0,
                      keepdims=True) * (1.0 / (8 * N))
        var = ex2 - mu * mu
        g = (h_ref[...] - mu) / jnp.sqrt(var + 1e-5) * gam_ref[...] \
            + bet_ref[...]
        g = jnp.maximum(g, 0.0)
        dr = drep_ref[...]
        for k in range(K):
            vk = jnp.dot(g, ci_ref[k], precision=HIGHEST)
            rk = jnp.dot(g, cr_ref[0, k], precision=HIGHEST) + cb_ref[0, k]
            vt_ref[k] = dr * jnp.broadcast_to(vk, (GB, HID))
            r2_ref[k] = jnp.broadcast_to(rk, (GB, HID))

    return pl.pallas_call(
        body,
        grid=(GRID,),
        in_specs=[
            pl.BlockSpec((GB, HID), lambda i: (i, 0)),
            _full_spec((GRID, 8, HID)),
            _full_spec((GRID, 8, HID)),
            _full_spec((1, HID)),
            _full_spec((1, HID)),
            pl.BlockSpec((GB, HID), lambda i: (i, 0)),
            _full_spec((K, HID, 1)),
            _full_spec((1, K, HID, 1)),
            _full_spec((1, K, 1, 1)),
        ],
        out_specs=[
            pl.BlockSpec((K, GB, HID), lambda i: (0, i, 0)),
            pl.BlockSpec((K, GB, HID), lambda i: (0, i, 0)),
        ],
        out_shape=[_sds((K, NPAD, HID)), _sds((K, NPAD, HID))],
    )(h, sums, sqs, gam, bet, drep, c2_init, c2_root, c2_bias)


def _t5(q2, r2, dinv2, c2_w):
    def body(q_ref, r2_ref, dinv2_ref, cw_ref, vn_ref):
        d = dinv2_ref[...]
        for k in range(K):
            o = d * (q_ref[0, k] + q_ref[1, k]) + r2_ref[k]
            vn_ref[k] = d * o * cw_ref[0, k]

    return pl.pallas_call(
        body,
        grid=(GRID,),
        in_specs=[
            pl.BlockSpec((NC, K, SB, 128), lambda i: (0, 0, i, 0)),
            pl.BlockSpec((K, SB, 128), lambda i: (0, i, 0)),
            pl.BlockSpec((SB, 128), lambda i: (i, 0)),
            _full_spec((1, K, 1, 1)),
        ],
        out_specs=[pl.BlockSpec((K, SB, 128), lambda i: (0, i, 0))],
        out_shape=[_sds((K, R128, 128))],
    )(q2, r2, dinv2, c2_w)[0]


def _t6(q2, r2, dinv2):
    def body(q_ref, r2_ref, dinv2_ref, y_ref):
        d = dinv2_ref[...]
        acc = jnp.zeros((SB, 128), F32)
        for k in range(K):
            acc = acc + d * (q_ref[0, k] + q_ref[1, k]) + r2_ref[k]
        y_ref[...] = jax.nn.sigmoid(acc * (1.0 / K))

    return pl.pallas_call(
        body,
        grid=(GRID,),
        in_specs=[
            pl.BlockSpec((NC, K, SB, 128), lambda i: (0, 0, i, 0)),
            pl.BlockSpec((K, SB, 128), lambda i: (0, i, 0)),
            pl.BlockSpec((SB, 128), lambda i: (i, 0)),
        ],
        out_specs=[pl.BlockSpec((SB, 128), lambda i: (i, 0))],
        out_shape=[_sds((R128, 128))],
    )(q2, r2, dinv2)[0]


def kernel(x, edge_index, edge_attr, c1_init, c1_w, c1_root, c1_bias, bn_g,
           bn_b, c2_init, c2_w, c2_root, c2_bias):
    src = edge_index[0]
    dst = edge_index[1]
    ew = edge_attr.astype(F32)
    npad_e = EPAD - E
    srcp = jnp.concatenate([src, jnp.zeros((npad_e,), jnp.int32)])
    srcp = srcp.reshape(ER, 128)
    dstp = jnp.concatenate([dst, jnp.full((npad_e,), NPAD - 1, jnp.int32)])
    dstp = dstp.reshape(ER, 128)
    ewp = jnp.concatenate([ew, jnp.zeros((npad_e,), F32)]).reshape(ER, 128)
    xp = jnp.zeros((NPAD,), F32).at[:N].set(x[:, 0])
    x2 = xp.reshape(R128, 128)

    degp = _deg_fn()(dst, ew)
    dinv2, xt2 = _t1(degp.reshape(NC, R128, 128), x2)
    drep = jnp.broadcast_to(dinv2.reshape(NPAD)[:, None], (NPAD, HID))
    xrep = jnp.broadcast_to(xp[:, None], (NPAD, HID))

    sp = _spmv_fn(1)(src, dst, ew, xt2.reshape(NPAD))
    a2 = _t2a(sp.reshape(NC, 1, R128, 128), dinv2)
    arep = jnp.broadcast_to(a2.reshape(NPAD)[:, None], (NPAD, HID))
    Ht = _t2(arep, drep, xrep, c1_init, c1_w, c1_root, c1_bias)

    for it in range(1, NUM_LAYERS):
        parts = [_spmm_fn()(srcp, dstp, ewp, Ht[k]) for k in range(K)]
        if it < NUM_LAYERS - 1:
            Ht = _t3(parts[0], parts[1], parts[2], drep, xrep,
                     c1_w, c1_root, c1_bias)
        else:
            h, sums, sqs = _t3c(parts[0], parts[1], parts[2], drep, xrep,
                                c1_root, c1_bias)

    vt_rep, r2_rep = _t4(h, sums, sqs, bn_g.reshape(1, HID),
                         bn_b.reshape(1, HID), drep,
                         c2_init, c2_root, c2_bias)
    vflat = vt_rep[:, :, 0]
    r2 = r2_rep[:, :, 0].reshape(K, R128, 128)

    for t in range(NUM_LAYERS):
        q = _spmv_fn(K)(src, dst, ew, vflat[0], vflat[1], vflat[2])
        q2 = q.reshape(NC, K, R128, 128)
        if t < NUM_LAYERS - 1:
            vflat = _t5(q2, r2, dinv2, c2_w).reshape(K, NPAD)
        else:
            y2 = _t6(q2, r2, dinv2)

    return y2.reshape(NPAD)[:N].reshape(N, 1)


# speedup vs baseline: 126.9540x; 1.8716x over previous
import functools

import jax
import jax.numpy as jnp
from jax import lax
from jax.experimental import pallas as pl
from jax.experimental.pallas import tpu as pltpu
from jax.experimental.pallas import tpu_sc as plsc

N = 100000
E = 1600000
HID = 16
K = 3
NUM_LAYERS = 4

NPAD = 102400
GB = 4096
GRID = NPAD // GB
R128 = NPAD // 128
SB = GB // 128
NC = 2
NS = 16
TR = NPAD // NS
E2 = E // NC
TE = E2 // NS
W = 80
NWIN = TE // W
EPAD = 1638400
ER = EPAD // 128
ER_SC = ER // NC
ER_TILE = ER_SC // NS
CH = 8
NCHUNK = ER_TILE // CH
F32 = jnp.float32
HIGHEST = lax.Precision.HIGHEST


def _mesh():
    return plsc.VectorSubcoreMesh(core_axis_name="c", subcore_axis_name="s",
                                  num_cores=NC, num_subcores=NS)


ZB = 128


def _zero_zb2(zb):
    def z(i, _):
        zb[i, :] = jnp.zeros((HID,), F32)
        return 0
    lax.fori_loop(0, ZB, z, 0)


def _zero_zb1(zb):
    def z(i, _):
        zb[pl.ds(i * 16, 16)] = jnp.zeros((16,), F32)
        return 0
    lax.fori_loop(0, ZB // 16, z, 0)


@functools.lru_cache(maxsize=None)
def _deg_fn():
    def body(dst_h, ew_h, out_h, dstb, ewb, zb, acc, isem, ssem):
        c = lax.axis_index("c")
        s = lax.axis_index("s")
        _zero_zb1(zb)
        for r in range(TR // ZB):
            pltpu.sync_copy(zb, acc.at[pl.ds(s * TR + r * ZB, ZB)])
        plsc.subcore_barrier()
        base = c * ER_SC + s * ER_TILE

        def idx_issue(q, p):
            rows = pl.ds(base + q * CH, CH)
            pltpu.async_copy(dst_h.at[rows], dstb.at[p], isem)
            pltpu.async_copy(ew_h.at[rows], ewb.at[p], isem)

        idx_issue(0, 0)

        def chunk(q, _):
            p = q % 2
            pltpu.make_async_copy(dst_h.at[pl.ds(0, CH)], dstb.at[p],
                                  isem).wait()
            pltpu.make_async_copy(ew_h.at[pl.ds(0, CH)], ewb.at[p],
                                  isem).wait()

            @pl.when(q > 0)
            def _drain_prev():
                pltpu.make_async_copy(ew_h.at[pl.ds(0, CH)], ewb.at[1 - p],
                                      ssem).wait()

            @pl.when(q < NCHUNK - 1)
            def _issue_next_idx():
                idx_issue(q + 1, 1 - p)

            for sub in range(CH):
                pltpu.async_copy(ewb.at[p, sub], acc.at[dstb.at[p, sub]],
                                 ssem, add=True)
            return 0

        lax.fori_loop(0, NCHUNK, chunk, 0)
        pltpu.make_async_copy(ew_h.at[pl.ds(0, CH)],
                              ewb.at[(NCHUNK - 1) % 2], ssem).wait()
        plsc.subcore_barrier()
        sl = pl.ds(s * TR, TR)
        pltpu.sync_copy(acc.at[sl], out_h.at[c, sl])

    return pl.kernel(
        body,
        out_type=jax.ShapeDtypeStruct((NC, NPAD), F32),
        mesh=_mesh(),
        compiler_params=pltpu.CompilerParams(use_tc_tiling_on_sc=False),
        scratch_types=[
            pltpu.VMEM((2, CH, 128), jnp.int32),
            pltpu.VMEM((2, CH, 128), F32),
            pltpu.VMEM((ZB,), F32),
            pltpu.VMEM_SHARED((NPAD,), F32),
            pltpu.SemaphoreType.DMA,
            pltpu.SemaphoreType.DMA,
        ],
    )


@functools.lru_cache(maxsize=None)
def _spmm_fn():
    def body(src_h, dst_h, ew_h, hk_h, out_h, srcb, dstb, ewb, upd, zb,
             acc, isem, gsem0, gsem1, ssem0, ssem1):
        c = lax.axis_index("c")
        s = lax.axis_index("s")
        _zero_zb2(zb)
        for r in range(TR // ZB):
            pltpu.sync_copy(zb, acc.at[pl.ds(s * TR + r * ZB, ZB)])
        plsc.subcore_barrier()
        base = c * ER_SC + s * ER_TILE
        gsem = (gsem0, gsem1)
        ssem = (ssem0, ssem1)

        def idx_issue(q, p):
            rows = pl.ds(base + q * CH, CH)
            pltpu.async_copy(src_h.at[rows], srcb.at[p], isem)
            pltpu.async_copy(dst_h.at[rows], dstb.at[p], isem)
            pltpu.async_copy(ew_h.at[rows], ewb.at[p], isem)

        def idx_wait(p):
            pltpu.make_async_copy(src_h.at[pl.ds(0, CH)], srcb.at[p],
                                  isem).wait()
            pltpu.make_async_copy(src_h.at[pl.ds(0, CH)], dstb.at[p],
                                  isem).wait()
            pltpu.make_async_copy(ew_h.at[pl.ds(0, CH)], ewb.at[p],
                                  isem).wait()

        def sdrain(u):
            pltpu.make_async_copy(hk_h.at[pl.ds(0, 128)], upd.at[u],
                                  ssem[u]).wait()

        idx_issue(0, 0)

        def chunk(q, _):
            p = q % 2
            idx_wait(p)

            @pl.when(q < NCHUNK - 1)
            def _issue_next_idx():
                idx_issue(q + 1, 1 - p)

            @pl.when(q > 0)
            def _drain_prev_s0():
                sdrain(0)

            pltpu.async_copy(hk_h.at[srcb.at[p, 0]], upd.at[0], gsem0)
            for sub in range(CH):
                u = sub % 2
                if sub == 0:
                    @pl.when(q > 0)
                    def _drain_prev_s1():
                        sdrain(1)
                elif sub < CH - 1:
                    sdrain(1 - u)
                if sub < CH - 1:
                    pltpu.async_copy(hk_h.at[srcb.at[p, sub + 1]],
                                     upd.at[1 - u], gsem[1 - u])
                pltpu.make_async_copy(hk_h.at[pl.ds(0, 128)], upd.at[u],
                                      gsem[u]).wait()

                def m_body(j, _):
                    ewv = ewb[p, sub, pl.ds(j * 16, 16)]
                    for t in range(16):
                        jj = j * 16 + t
                        upd[u, jj, :] = upd[u, jj, :] * ewv[t]
                    return 0

                lax.fori_loop(0, 8, m_body, 0)
                pltpu.async_copy(upd.at[u], acc.at[dstb.at[p, sub]],
                                 ssem[u], add=True)
            return 0

        lax.fori_loop(0, NCHUNK, chunk, 0)
        sdrain(0)
        sdrain(1)
        plsc.subcore_barrier()
        sl = pl.ds(s * TR, TR)
        pltpu.sync_copy(acc.at[sl], out_h.at[c, sl])

    return pl.kernel(
        body,
        out_type=jax.ShapeDtypeStruct((NC, NPAD, HID), F32),
        mesh=_mesh(),
        compiler_params=pltpu.CompilerParams(use_tc_tiling_on_sc=False),
        scratch_types=[
            pltpu.VMEM((2, CH, 128), jnp.int32),
            pltpu.VMEM((2, CH, 128), jnp.int32),
            pltpu.VMEM((2, CH, 128), F32),
            pltpu.VMEM((2, 128, HID), F32),
            pltpu.VMEM((ZB, HID), F32),
            pltpu.VMEM_SHARED((NPAD, HID), F32),
            pltpu.SemaphoreType.DMA,
            pltpu.SemaphoreType.DMA,
            pltpu.SemaphoreType.DMA,
            pltpu.SemaphoreType.DMA,
            pltpu.SemaphoreType.DMA,
        ],
    )


@functools.lru_cache(maxsize=None)
def _spmv_fn(C):
    def body(src_h, dst_h, ew_h, *rest):
        xs = rest[:C]
        out_h = rest[C]
        srcb, dstb, ewb, upd, zb = rest[C + 1:C + 6]
        vsh = rest[C + 6:C + 6 + C]
        acc = rest[C + 6 + C:C + 6 + 2 * C]
        isem, gsem0, gsem1, ssem0, ssem1 = rest[C + 6 + 2 * C:]
        c = lax.axis_index("c")
        s = lax.axis_index("s")
        sl = pl.ds(s * TR, TR)
        for ch in range(C):
            pltpu.sync_copy(xs[ch].at[sl], vsh[ch].at[sl])
        _zero_zb1(zb)
        for ch in range(C):
            for r in range(TR // ZB):
                pltpu.sync_copy(zb, acc[ch].at[pl.ds(s * TR + r * ZB, ZB)])
        plsc.subcore_barrier()
        base = c * ER_SC + s * ER_TILE
        gsem = (gsem0, gsem1)
        ssem = (ssem0, ssem1)

        def idx_issue(q, p):
            rows = pl.ds(base + q * CH, CH)
            pltpu.async_copy(src_h.at[rows], srcb.at[p], isem)
            pltpu.async_copy(dst_h.at[rows], dstb.at[p], isem)
            pltpu.async_copy(ew_h.at[rows], ewb.at[p], isem)

        def idx_wait(p):
            pltpu.make_async_copy(src_h.at[pl.ds(0, CH)], srcb.at[p],
                                  isem).wait()
            pltpu.make_async_copy(src_h.at[pl.ds(0, CH)], dstb.at[p],
                                  isem).wait()
            pltpu.make_async_copy(ew_h.at[pl.ds(0, CH)], ewb.at[p],
                                  isem).wait()

        def g_issue(p, sub, u):
            for ch in range(C):
                pltpu.async_copy(vsh[ch].at[srcb.at[p, sub]],
                                 upd.at[u, ch], gsem[u])

        def g_wait(u):
            pltpu.make_async_copy(ew_h.at[pl.ds(0, C)], upd.at[u],
                                  gsem[u]).wait()

        def sdrain(u):
            pltpu.make_async_copy(ew_h.at[pl.ds(0, C)], upd.at[u],
                                  ssem[u]).wait()

        idx_issue(0, 0)

        def chunk(q, _):
            p = q % 2
            idx_wait(p)

            @pl.when(q < NCHUNK - 1)
            def _issue_next_idx():
                idx_issue(q + 1, 1 - p)

            @pl.when(q > 0)
            def _drain_prev_s0():
                sdrain(0)

            g_issue(p, 0, 0)
            for sub in range(CH):
                u = sub % 2
                if sub == 0:
                    @pl.when(q > 0)
                    def _drain_prev_s1():
                        sdrain(1)
                elif sub < CH - 1:
                    sdrain(1 - u)
                if sub < CH - 1:
                    g_issue(p, sub + 1, 1 - u)
                g_wait(u)
                for ch in range(C):
                    def m_body(j, _, _ch=ch):
                        s16 = pl.ds(j * 16, 16)
                        upd[u, _ch, s16] = upd[u, _ch, s16] \
                            * ewb[p, sub, s16]
                        return 0
                    lax.fori_loop(0, 8, m_body, 0)
                for ch in range(C):
                    pltpu.async_copy(upd.at[u, ch],
                                     acc[ch].at[dstb.at[p, sub]],
                                     ssem[u], add=True)
            return 0

        lax.fori_loop(0, NCHUNK, chunk, 0)
        sdrain(0)
        sdrain(1)
        plsc.subcore_barrier()
        for ch in range(C):
            pltpu.sync_copy(acc[ch].at[sl], out_h.at[c, ch, sl])

    return pl.kernel(
        body,
        out_type=jax.ShapeDtypeStruct((NC, C, NPAD), F32),
        mesh=_mesh(),
        compiler_params=pltpu.CompilerParams(use_tc_tiling_on_sc=False),
        scratch_types=(
            [pltpu.VMEM((2, CH, 128), jnp.int32),
             pltpu.VMEM((2, CH, 128), jnp.int32),
             pltpu.VMEM((2, CH, 128), F32),
             pltpu.VMEM((2, C, 128), F32),
             pltpu.VMEM((ZB,), F32)]
            + [pltpu.VMEM_SHARED((NPAD,), F32) for _ in range(2 * C)]
            + [pltpu.SemaphoreType.DMA] * 5
        ),
    )


def _sds(shape):
    return jax.ShapeDtypeStruct(shape, F32)


def _full_spec(shape):
    rank = len(shape)
    return pl.BlockSpec(shape, lambda i, _r=rank: (0,) * _r)


def _t1(degp2, x2):
    def body(degp_ref, x_ref, dinv2_ref, xt2_ref):
        deg = degp_ref[0] + degp_ref[1]
        pos = deg > 0
        safe = jnp.where(pos, deg, 1.0)
        dinv = jnp.where(pos, lax.rsqrt(safe), 0.0)
        dinv2_ref[...] = dinv
        xt2_ref[...] = dinv * x_ref[...]

    return pl.pallas_call(
        body,
        grid=(GRID,),
        in_specs=[
            pl.BlockSpec((NC, SB, 128), lambda i: (0, i, 0)),
            pl.BlockSpec((SB, 128), lambda i: (i, 0)),
        ],
        out_specs=[
            pl.BlockSpec((SB, 128), lambda i: (i, 0)),
            pl.BlockSpec((SB, 128), lambda i: (i, 0)),
        ],
        out_shape=[_sds((R128, 128)), _sds((R128, 128))],
    )(degp2, x2)


def _t2a(sp2, dinv2):
    def body(sp_ref, dinv2_ref, a_ref):
        a_ref[...] = dinv2_ref[...] * (sp_ref[0, 0] + sp_ref[1, 0])

    return pl.pallas_call(
        body,
        grid=(GRID,),
        in_specs=[
            pl.BlockSpec((NC, 1, SB, 128), lambda i: (0, 0, i, 0)),
            pl.BlockSpec((SB, 128), lambda i: (i, 0)),
        ],
        out_specs=[pl.BlockSpec((SB, 128), lambda i: (i, 0))],
        out_shape=[_sds((R128, 128))],
    )(sp2, dinv2)[0]


def _t2(arep, drep, xrep, c1_init, c1_w, c1_root, c1_bias):
    def body(arep_ref, drep_ref, xrep_ref, ci_ref, cw_ref, cr_ref,
             cb_ref, out_ref):
        ar = arep_ref[...]
        xr = xrep_ref[...]
        dr = drep_ref[...]
        for k in range(K):
            o = ar * ci_ref[k] + xr * cr_ref[0, k] + cb_ref[0, k]
            o = jnp.maximum(o, 0.0)
            hk = jnp.dot(o, cw_ref[0, k], precision=HIGHEST)
            out_ref[k] = dr * hk

    return pl.pallas_call(
        body,
        grid=(GRID,),
        in_specs=[
            pl.BlockSpec((GB, HID), lambda i: (i, 0)),
            pl.BlockSpec((GB, HID), lambda i: (i, 0)),
            pl.BlockSpec((GB, HID), lambda i: (i, 0)),
            _full_spec((K, 1, HID)),
            _full_spec((1, K, HID, HID)),
            _full_spec((1, K, 1, HID)),
            _full_spec((1, K, 1, HID)),
        ],
        out_specs=[pl.BlockSpec((K, GB, HID), lambda i: (0, i, 0))],
        out_shape=[_sds((K, NPAD, HID))],
    )(arep, drep, xrep, c1_init, c1_w, c1_root, c1_bias)[0]


def _t3(p0, p1, p2, drep, xrep, c1_w, c1_root, c1_bias):
    def body(p0_ref, p1_ref, p2_ref, drep_ref, xrep_ref, cw_ref, cr_ref,
             cb_ref, out_ref):
        ps = (p0_ref, p1_ref, p2_ref)
        xr = xrep_ref[...]
        dr = drep_ref[...]
        for k in range(K):
            agg = dr * (ps[k][0] + ps[k][1])
            o = agg + xr * cr_ref[0, k] + cb_ref[0, k]
            o = jnp.maximum(o, 0.0)
            out_ref[k] = dr * jnp.dot(o, cw_ref[0, k], precision=HIGHEST)

    part_spec = pl.BlockSpec((NC, GB, HID), lambda i: (0, i, 0))
    return pl.pallas_call(
        body,
        grid=(GRID,),
        in_specs=[
            part_spec, part_spec, part_spec,
            pl.BlockSpec((GB, HID), lambda i: (i, 0)),
            pl.BlockSpec((GB, HID), lambda i: (i, 0)),
            _full_spec((1, K, HID, HID)),
            _full_spec((1, K, 1, HID)),
            _full_spec((1, K, 1, HID)),
        ],
        out_specs=[pl.BlockSpec((K, GB, HID), lambda i: (0, i, 0))],
        out_shape=[_sds((K, NPAD, HID))],
    )(p0, p1, p2, drep, xrep, c1_w, c1_root, c1_bias)[0]


def _t3c(p0, p1, p2, drep, xrep, c1_root, c1_bias):
    def body(p0_ref, p1_ref, p2_ref, drep_ref, xrep_ref, cr_ref, cb_ref,
             h_ref, sum_ref, sq_ref):
        i = pl.program_id(0)
        ps = (p0_ref, p1_ref, p2_ref)
        xr = xrep_ref[...]
        dr = drep_ref[...]
        acc = jnp.zeros((GB, HID), F32)
        for k in range(K):
            o = dr * (ps[k][0] + ps[k][1]) + xr * cr_ref[0, k] + cb_ref[0, k]
            acc = acc + jnp.maximum(o, 0.0)
        h = acc * (1.0 / K)
        h_ref[...] = h
        node = i * GB + lax.broadcasted_iota(jnp.int32, (GB, 1), 0)
        hm = jnp.where(node < N, h, 0.0)
        sum_ref[...] = jnp.broadcast_to(
            jnp.sum(hm, axis=0, keepdims=True), (8, HID))[None]
        sq_ref[...] = jnp.broadcast_to(
            jnp.sum(hm * hm, axis=0, keepdims=True), (8, HID))[None]

    part_spec = pl.BlockSpec((NC, GB, HID), lambda i: (0, i, 0))
    return pl.pallas_call(
        body,
        grid=(GRID,),
        in_specs=[
            part_spec, part_spec, part_spec,
            pl.BlockSpec((GB, HID), lambda i: (i, 0)),
            pl.BlockSpec((GB, HID), lambda i: (i, 0)),
            _full_spec((1, K, 1, HID)),
            _full_spec((1, K, 1, HID)),
        ],
        out_specs=[
            pl.BlockSpec((GB, HID), lambda i: (i, 0)),
            pl.BlockSpec((1, 8, HID), lambda i: (i, 0, 0)),
            pl.BlockSpec((1, 8, HID), lambda i: (i, 0, 0)),
        ],
        out_shape=[_sds((NPAD, HID)), _sds((GRID, 8, HID)),
                   _sds((GRID, 8, HID))],
    )(p0, p1, p2, drep, xrep, c1_root, c1_bias)


def _t4(h, sums, sqs, gam, bet, drep, c2_init, c2_root, c2_bias):
    def body(h_ref, sums_ref, sqs_ref, gam_ref, bet_ref, drep_ref, ci_ref,
             cr_ref, cb_ref, vt_ref, r2_ref):
        mu = jnp.sum(jnp.sum(sums_ref[...], axis=0), axis=0,
                     keepdims=True) * (1.0 / (8 * N))
        ex2 = jnp.sum(jnp.sum(sqs_ref[...], axis=0), axis=0,
                      keepdims=True) * (1.0 / (8 * N))
        var = ex2 - mu * mu
        g = (h_ref[...] - mu) / jnp.sqrt(var + 1e-5) * gam_ref[...] \
            + bet_ref[...]
        g = jnp.maximum(g, 0.0)
        dr = drep_ref[...]
        for k in range(K):
            vk = jnp.dot(g, ci_ref[k], precision=HIGHEST)
            rk = jnp.dot(g, cr_ref[0, k], precision=HIGHEST) + cb_ref[0, k]
            vt_ref[k] = dr * jnp.broadcast_to(vk, (GB, HID))
            r2_ref[k] = jnp.broadcast_to(rk, (GB, HID))

    return pl.pallas_call(
        body,
        grid=(GRID,),
        in_specs=[
            pl.BlockSpec((GB, HID), lambda i: (i, 0)),
            _full_spec((GRID, 8, HID)),
            _full_spec((GRID, 8, HID)),
            _full_spec((1, HID)),
            _full_spec((1, HID)),
            pl.BlockSpec((GB, HID), lambda i: (i, 0)),
            _full_spec((K, HID, 1)),
            _full_spec((1, K, HID, 1)),
            _full_spec((1, K, 1, 1)),
        ],
        out_specs=[
            pl.BlockSpec((K, GB, HID), lambda i: (0, i, 0)),
            pl.BlockSpec((K, GB, HID), lambda i: (0, i, 0)),
        ],
        out_shape=[_sds((K, NPAD, HID)), _sds((K, NPAD, HID))],
    )(h, sums, sqs, gam, bet, drep, c2_init, c2_root, c2_bias)


def _t5(q2, r2, dinv2, c2_w):
    def body(q_ref, r2_ref, dinv2_ref, cw_ref, vn_ref):
        d = dinv2_ref[...]
        for k in range(K):
            o = d * (q_ref[0, k] + q_ref[1, k]) + r2_ref[k]
            vn_ref[k] = d * o * cw_ref[0, k]

    return pl.pallas_call(
        body,
        grid=(GRID,),
        in_specs=[
            pl.BlockSpec((NC, K, SB, 128), lambda i: (0, 0, i, 0)),
            pl.BlockSpec((K, SB, 128), lambda i: (0, i, 0)),
            pl.BlockSpec((SB, 128), lambda i: (i, 0)),
            _full_spec((1, K, 1, 1)),
        ],
        out_specs=[pl.BlockSpec((K, SB, 128), lambda i: (0, i, 0))],
        out_shape=[_sds((K, R128, 128))],
    )(q2, r2, dinv2, c2_w)[0]


def _t6(q2, r2, dinv2):
    def body(q_ref, r2_ref, dinv2_ref, y_ref):
        d = dinv2_ref[...]
        acc = jnp.zeros((SB, 128), F32)
        for k in range(K):
            acc = acc + d * (q_ref[0, k] + q_ref[1, k]) + r2_ref[k]
        y_ref[...] = jax.nn.sigmoid(acc * (1.0 / K))

    return pl.pallas_call(
        body,
        grid=(GRID,),
        in_specs=[
            pl.BlockSpec((NC, K, SB, 128), lambda i: (0, 0, i, 0)),
            pl.BlockSpec((K, SB, 128), lambda i: (0, i, 0)),
            pl.BlockSpec((SB, 128), lambda i: (i, 0)),
        ],
        out_specs=[pl.BlockSpec((SB, 128), lambda i: (i, 0))],
        out_shape=[_sds((R128, 128))],
    )(q2, r2, dinv2)[0]


def kernel(x, edge_index, edge_attr, c1_init, c1_w, c1_root, c1_bias, bn_g,
           bn_b, c2_init, c2_w, c2_root, c2_bias):
    src = edge_index[0]
    dst = edge_index[1]
    ew = edge_attr.astype(F32)
    npad_e = EPAD - E
    srcp = jnp.concatenate([src, jnp.zeros((npad_e,), jnp.int32)])
    srcp = srcp.reshape(ER, 128)
    dstp = jnp.concatenate([dst, jnp.full((npad_e,), NPAD - 1, jnp.int32)])
    dstp = dstp.reshape(ER, 128)
    ewp = jnp.concatenate([ew, jnp.zeros((npad_e,), F32)]).reshape(ER, 128)
    xp = jnp.zeros((NPAD,), F32).at[:N].set(x[:, 0])
    x2 = xp.reshape(R128, 128)

    degp = _deg_fn()(dstp, ewp)
    dinv2, xt2 = _t1(degp.reshape(NC, R128, 128), x2)
    drep = jnp.broadcast_to(dinv2.reshape(NPAD)[:, None], (NPAD, HID))
    xrep = jnp.broadcast_to(xp[:, None], (NPAD, HID))

    sp = _spmv_fn(1)(srcp, dstp, ewp, xt2.reshape(NPAD))
    a2 = _t2a(sp.reshape(NC, 1, R128, 128), dinv2)
    arep = jnp.broadcast_to(a2.reshape(NPAD)[:, None], (NPAD, HID))
    Ht = _t2(arep, drep, xrep, c1_init, c1_w, c1_root, c1_bias)

    for it in range(1, NUM_LAYERS):
        parts = [_spmm_fn()(srcp, dstp, ewp, Ht[k]) for k in range(K)]
        if it < NUM_LAYERS - 1:
            Ht = _t3(parts[0], parts[1], parts[2], drep, xrep,
                     c1_w, c1_root, c1_bias)
        else:
            h, sums, sqs = _t3c(parts[0], parts[1], parts[2], drep, xrep,
                                c1_root, c1_bias)

    vt_rep, r2_rep = _t4(h, sums, sqs, bn_g.reshape(1, HID),
                         bn_b.reshape(1, HID), drep,
                         c2_init, c2_root, c2_bias)
    vflat = vt_rep[:, :, 0]
    r2 = r2_rep[:, :, 0].reshape(K, R128, 128)

    for t in range(NUM_LAYERS):
        q = _spmv_fn(K)(srcp, dstp, ewp, vflat[0], vflat[1], vflat[2])
        q2 = q.reshape(NC, K, R128, 128)
        if t < NUM_LAYERS - 1:
            vflat = _t5(q2, r2, dinv2, c2_w).reshape(K, NPAD)
        else:
            y2 = _t6(q2, r2, dinv2)

    return y2.reshape(NPAD)[:N].reshape(N, 1)
